# baseline (speedup 1.0000x reference)
import functools

import jax
import jax.numpy as jnp
from jax import lax
from jax.experimental import pallas as pl
from jax.experimental.pallas import tpu as pltpu
from jax.experimental.pallas import tpu_sc as plsc

PAD_ID = 0

_NC = 2
_NS = 16
_NW = _NC * _NS
_L = 16
_BB = 16
_SB = 40


@functools.cache
def _make_gather(v, d, max_pos, batch, seq):
  assert batch % (_NW * _BB) == 0 and seq % _SB == 0 and _SB % 8 == 0
  bbp = _BB + 1
  b_per_w = batch // _NW
  nb = b_per_w // _BB
  ns = seq // _SB
  ntok = _BB * _SB
  mesh = plsc.VectorSubcoreMesh(core_axis_name="c", subcore_axis_name="s")

  @functools.partial(
      pl.kernel,
      out_type=jax.ShapeDtypeStruct((seq * d, batch), jnp.float32),
      mesh=mesh,
      scratch_types=[
          pltpu.VMEM((_BB, _SB), jnp.int32),
          pltpu.VMEM((ntok, d), jnp.float32),
          pltpu.VMEM((_SB * d, bbp), jnp.float32),
          pltpu.VMEM((max_pos, d), jnp.float32),
          pltpu.SemaphoreType.DMA,
          pltpu.SemaphoreType.DMA,
      ],
      compiler_params=pltpu.CompilerParams(
          use_tc_tiling_on_sc=False, needs_layout_passes=False
      ),
  )
  def gather_k(tok_hbm, table_hbm, pos_hbm, out_hbm,
               idxb, rows, outs, posbuf, gs, ws):
    wid = lax.axis_index("s") * _NC + lax.axis_index("c")
    b_base = wid * b_per_w
    iota16 = lax.iota(jnp.int32, _L)

    pltpu.sync_copy(pos_hbm, posbuf)

    def block(bi, si):
      b0 = b_base + bi * _BB
      s0 = si * _SB
      pltpu.sync_copy(
          tok_hbm.at[pl.ds(b0, _BB), pl.ds(s0, _SB)], idxb
      )
      cps = [
          pltpu.async_copy(
              table_hbm.at[idxb.at[i]], rows.at[pl.ds(i * _SB, _SB)], gs
          )
          for i in range(_BB)
      ]
      for cp in cps:
        cp.wait()

      @plsc.parallel_loop(0, _SB, unroll=2)
      def srow(sl):
        tvec = plsc.load_gather(idxb, [iota16, jnp.full((_L,), sl, jnp.int32)])
        mvec = jnp.where(tvec != PAD_ID, 1.0, 0.0).astype(jnp.float32)
        pos4 = [posbuf[s0 + sl, pl.ds(jg * _L, _L)] for jg in range(d // _L)]
        rbase = sl * d
        for b in range(_BB):
          m = jnp.full((_L,), mvec[b])
          trow = b * _SB + sl
          for jg in range(d // _L):
            val = rows[trow, pl.ds(jg * _L, _L)] * m + pos4[jg]
            plsc.store_scatter(
                outs,
                [jnp.full((_L,), rbase + jg * _L, jnp.int32) + iota16,
                 jnp.full((_L,), b, jnp.int32)],
                val,
            )

      pltpu.sync_copy(
          outs.at[:, pl.ds(0, _BB)],
          out_hbm.at[pl.ds(s0 * d, _SB * d), pl.ds(b0, _BB)],
      )

    def outer(i):
      block(lax.div(i, ns), lax.rem(i, ns))

    pl.loop(0, nb * ns)(outer)

  return gather_k


def kernel(tokens, embed_weight, pos_weight):
  batch, seq = tokens.shape
  v, d = embed_weight.shape
  max_pos = pos_weight.shape[0]
  tok32 = tokens.astype(jnp.int32)
  out_t = _make_gather(v, d, max_pos, batch, seq)(
      tok32, embed_weight, pos_weight
  )
  return out_t.reshape(seq, d, batch).transpose(2, 0, 1)


# speedup vs baseline: 1.1234x; 1.1234x over previous
import functools

import jax
import jax.numpy as jnp
from jax import lax
from jax.experimental import pallas as pl
from jax.experimental.pallas import tpu as pltpu
from jax.experimental.pallas import tpu_sc as plsc

PAD_ID = 0

_NC = 2
_NS = 16
_NW = _NC * _NS
_L = 16
_BB = 16
_SB = 40


@functools.cache
def _make_gather(v, d, max_pos, batch, seq):
  assert batch % (_NW * _BB) == 0 and seq % _SB == 0 and _SB % 8 == 0
  bbp = _BB + 1
  b_per_w = batch // _NW
  nb = b_per_w // _BB
  ns = seq // _SB
  ntok = _BB * _SB
  mesh = plsc.VectorSubcoreMesh(core_axis_name="c", subcore_axis_name="s")

  @functools.partial(
      pl.kernel,
      out_type=jax.ShapeDtypeStruct((seq * d, batch), jnp.float32),
      mesh=mesh,
      scratch_types=[
          pltpu.VMEM((_BB, _SB), jnp.int32),
          pltpu.VMEM((ntok, d), jnp.float32),
          pltpu.VMEM((_SB * d, bbp), jnp.float32),
          pltpu.VMEM((max_pos, d), jnp.float32),
          pltpu.SemaphoreType.DMA,
          pltpu.SemaphoreType.DMA,
      ],
      compiler_params=pltpu.CompilerParams(
          use_tc_tiling_on_sc=False, needs_layout_passes=False
      ),
  )
  def gather_k(tok_hbm, table_hbm, pos_hbm, out_hbm,
               idxb, rows, outs, posbuf, gs, ws):
    wid = lax.axis_index("s") * _NC + lax.axis_index("c")
    b_base = wid * b_per_w
    iota16 = lax.iota(jnp.int32, _L)

    pltpu.sync_copy(pos_hbm, posbuf)

    def block(bi, si):
      b0 = b_base + bi * _BB
      s0 = si * _SB
      pltpu.sync_copy(
          tok_hbm.at[pl.ds(b0, _BB), pl.ds(s0, _SB)], idxb
      )
      cps = [
          pltpu.async_copy(
              table_hbm.at[idxb.at[i]], rows.at[pl.ds(i * _SB, _SB)], gs
          )
          for i in range(_BB)
      ]
      for cp in cps:
        cp.wait()

      @plsc.parallel_loop(0, _SB, unroll=4)
      def srow(sl):
        tvec = plsc.load_gather(idxb, [iota16, jnp.full((_L,), sl, jnp.int32)])
        mvec = jnp.where(tvec != PAD_ID, 1.0, 0.0).astype(jnp.float32)
        pos4 = [posbuf[s0 + sl, pl.ds(jg * _L, _L)] for jg in range(d // _L)]
        rbase = sl * d
        rvecs = [iota16 + (rbase + jg * _L) for jg in range(d // _L)]
        for b in range(_BB):
          m = jnp.full((_L,), mvec[b])
          cvec = jnp.full((_L,), b, jnp.int32)
          trow = b * _SB + sl
          for jg in range(d // _L):
            val = rows[trow, pl.ds(jg * _L, _L)] * m + pos4[jg]
            plsc.store_scatter(outs, [rvecs[jg], cvec], val)

      pltpu.sync_copy(
          outs.at[:, pl.ds(0, _BB)],
          out_hbm.at[pl.ds(s0 * d, _SB * d), pl.ds(b0, _BB)],
      )

    def outer(i):
      block(lax.div(i, ns), lax.rem(i, ns))

    pl.loop(0, nb * ns)(outer)

  return gather_k


def kernel(tokens, embed_weight, pos_weight):
  batch, seq = tokens.shape
  v, d = embed_weight.shape
  max_pos = pos_weight.shape[0]
  tok32 = tokens.astype(jnp.int32)
  out_t = _make_gather(v, d, max_pos, batch, seq)(
      tok32, embed_weight, pos_weight
  )
  return out_t.reshape(seq, d, batch).transpose(2, 0, 1)
